# baseline (speedup 1.0000x reference)
import jax
import jax.numpy as jnp
from jax.experimental import pallas as pl

_N = 4096
_BLK = 256
_GRID = _N // _BLK
_TL = 512

_SORT8 = (
    (0, 1), (2, 3), (4, 5), (6, 7),
    (0, 2), (1, 3), (4, 6), (5, 7),
    (1, 2), (5, 6),
    (0, 4), (1, 5), (2, 6), (3, 7),
    (2, 4), (3, 5),
    (1, 2), (3, 4), (5, 6),
)
_BITONIC8 = (
    (0, 4), (1, 5), (2, 6), (3, 7),
    (0, 2), (1, 3), (4, 6), (5, 7),
    (0, 1), (2, 3), (4, 5), (6, 7),
)


def _cmpex(lists, i, j):
    lo = jnp.minimum(lists[i], lists[j])
    hi = jnp.maximum(lists[i], lists[j])
    lists[i] = lo
    lists[j] = hi


def _top8_desc_ax0(prod):
    c, _ = prod.shape
    w = c // 8
    lists = [prod[t * w:(t + 1) * w, :] for t in range(8)]
    for i, j in _SORT8:
        _cmpex(lists, i, j)
    while w > 1:
        w //= 2
        a = [l[:w, :] for l in lists]
        b = [l[w:, :] for l in lists]
        lists = [jnp.maximum(a[i], b[7 - i]) for i in range(8)]
        for i, j in _BITONIC8:
            _cmpex(lists, i, j)
    return jnp.concatenate(lists[::-1], axis=0)


def _mm_body(a_ref, b_ref, o_ref):
    o_ref[...] = jnp.dot(a_ref[...], b_ref[...],
                         preferred_element_type=jnp.float32)


def _small_mm(a, b):
    n, _ = a.shape
    m = b.shape[1]
    return pl.pallas_call(
        _mm_body,
        out_shape=jax.ShapeDtypeStruct((n, m), jnp.float32),
    )(a, b)


def _row_mm(adj, b):
    m = b.shape[1]
    return pl.pallas_call(
        _mm_body,
        grid=(_GRID,),
        in_specs=[
            pl.BlockSpec((_BLK, _N), lambda i: (i, 0)),
            pl.BlockSpec((_N, m), lambda i: (0, 0)),
        ],
        out_specs=pl.BlockSpec((_BLK, m), lambda i: (i, 0)),
        out_shape=jax.ShapeDtypeStruct((_N, m), jnp.float32),
    )(adj, b)


def _row_mm_mask_body(adj_ref, b_ref, mask_ref, o_ref):
    o_ref[...] = jnp.dot(adj_ref[...], b_ref[...],
                         preferred_element_type=jnp.float32) * mask_ref[...]


def _row_mm_mask(adj, b, maskf):
    m = b.shape[1]
    return pl.pallas_call(
        _row_mm_mask_body,
        grid=(_GRID,),
        in_specs=[
            pl.BlockSpec((_BLK, _N), lambda i: (i, 0)),
            pl.BlockSpec((_N, m), lambda i: (0, 0)),
            pl.BlockSpec((_BLK, 1), lambda i: (i, 0)),
        ],
        out_specs=pl.BlockSpec((_BLK, m), lambda i: (i, 0)),
        out_shape=jax.ShapeDtypeStruct((_N, m), jnp.float32),
    )(adj, b, maskf)


def _topk_dual_body(adjt_ref, hcol_ref, htb_ref, ckt1_ref, ckt2_ref,
                    c0t1_ref, c0t2_ref, b1_ref, b2_ref, o1_ref, o2_ref):
    f = pl.program_id(1)

    @pl.when(f == 0)
    def _init():
        htb = htb_ref[...]
        o1_ref[...] = jnp.dot(c0t1_ref[...], htb,
                              preferred_element_type=jnp.float32) + b1_ref[...]
        o2_ref[...] = jnp.dot(c0t2_ref[...], htb,
                              preferred_element_type=jnp.float32) + b2_ref[...]

    adjt = adjt_ref[...]
    acc1 = o1_ref[...]
    acc2 = o2_ref[...]
    for k in range(4):
        prod = adjt * hcol_ref[k]
        t8 = _top8_desc_ax0(prod)
        acc1 = acc1 + jnp.dot(ckt1_ref[k], t8,
                              preferred_element_type=jnp.float32)
        acc2 = acc2 + jnp.dot(ckt2_ref[k], t8,
                              preferred_element_type=jnp.float32)
    o1_ref[...] = acc1
    o2_ref[...] = acc2


def _topk_tail_body(adjt_ref, hcol_ref, curtb_ref, part_ref, ckt_ref,
                    c0t_ref, o_ref):
    f = pl.program_id(1)

    @pl.when(f == 0)
    def _init():
        o_ref[...] = part_ref[...] + jnp.dot(
            c0t_ref[...], curtb_ref[...], preferred_element_type=jnp.float32)

    adjt = adjt_ref[...]
    acc = o_ref[...]
    for k in range(4):
        prod = adjt * hcol_ref[k]
        t8 = _top8_desc_ax0(prod)
        acc = acc + jnp.dot(ckt_ref[k], t8, preferred_element_type=jnp.float32)
    o_ref[...] = acc


def _topk_dual(adjt, ht, cw1, cw2, bias1, bias2):
    nfeat = ht.shape[0]
    hc3 = ht.reshape(nfeat, _N, 1)
    ckt1 = jnp.transpose(cw1[1:9], (1, 2, 0))
    ckt2 = jnp.transpose(cw2[1:9, :nfeat], (1, 2, 0))
    c0t1 = cw1[0].T
    c0t2 = cw2[0, :nfeat].T
    b1 = bias1.reshape(8, 1)
    b2 = bias2.reshape(8, 1)
    return pl.pallas_call(
        _topk_dual_body,
        grid=(_N // _TL, nfeat // 4),
        in_specs=[
            pl.BlockSpec((_N, _TL), lambda i, f: (0, i)),
            pl.BlockSpec((4, _N, 1), lambda i, f: (f, 0, 0)),
            pl.BlockSpec((nfeat, _TL), lambda i, f: (0, i)),
            pl.BlockSpec((4, 8, 8), lambda i, f: (f, 0, 0)),
            pl.BlockSpec((4, 8, 8), lambda i, f: (f, 0, 0)),
            pl.BlockSpec((8, nfeat), lambda i, f: (0, 0)),
            pl.BlockSpec((8, nfeat), lambda i, f: (0, 0)),
            pl.BlockSpec((8, 1), lambda i, f: (0, 0)),
            pl.BlockSpec((8, 1), lambda i, f: (0, 0)),
        ],
        out_specs=[
            pl.BlockSpec((8, _TL), lambda i, f: (0, i)),
            pl.BlockSpec((8, _TL), lambda i, f: (0, i)),
        ],
        out_shape=[
            jax.ShapeDtypeStruct((8, _N), jnp.float32),
            jax.ShapeDtypeStruct((8, _N), jnp.float32),
        ],
    )(adjt, hc3, ht, ckt1, ckt2, c0t1, c0t2, b1, b2)


def _topk_tail(adjt, curt, part2, cw2):
    nf2 = cw2.shape[1]
    hc3 = curt.reshape(8, _N, 1)
    ckt = jnp.transpose(cw2[1:9, nf2 - 8:], (1, 2, 0))
    c0t = cw2[0, nf2 - 8:].T
    return pl.pallas_call(
        _topk_tail_body,
        grid=(_N // _TL, 2),
        in_specs=[
            pl.BlockSpec((_N, _TL), lambda i, f: (0, i)),
            pl.BlockSpec((4, _N, 1), lambda i, f: (f, 0, 0)),
            pl.BlockSpec((8, _TL), lambda i, f: (0, i)),
            pl.BlockSpec((8, _TL), lambda i, f: (0, i)),
            pl.BlockSpec((4, 8, 8), lambda i, f: (f, 0, 0)),
            pl.BlockSpec((8, 8), lambda i, f: (0, 0)),
        ],
        out_specs=pl.BlockSpec((8, _TL), lambda i, f: (0, i)),
        out_shape=jax.ShapeDtypeStruct((8, _N), jnp.float32),
    )(adjt, hc3, curt, part2, ckt, c0t)


def _fuse_conv_weights(wa, wb, gamma):
    kwa, cin, _ = wa.shape
    kwb, _, cout = wb.shape
    cw = jnp.zeros((kwa + kwb - 1, cin, cout), jnp.float32)
    for u in range(kwb):
        for v in range(kwa):
            cw = cw.at[u + v].add(wa[v] @ wb[u])
    scale = gamma / jnp.sqrt(1.0 + 1e-3)
    return cw * scale[None, None, :]


def kernel(x, adj, mask, w1, c1a, c1b, g1, b1, c2a, c2b, g2, b2, w_out):
    cw1 = _fuse_conv_weights(c1a, c1b, g1)
    cw2 = _fuse_conv_weights(c2a, c2b, g2)
    maskf = mask.astype(jnp.float32).reshape(_N, 1)
    adjt = adj.T

    h1 = _row_mm(adj, _small_mm(x, w1))
    h1t = h1.T
    cur1t, part2 = _topk_dual(adjt, h1t, cw1, cw2, b1, b2)
    cur2t = _topk_tail(adjt, cur1t, part2, cw2)
    h3 = jnp.concatenate([h1, cur1t.T, cur2t.T], axis=1)
    return _row_mm_mask(adj, _small_mm(h3, w_out), maskf)
